# Initial kernel scaffold; baseline (speedup 1.0000x reference)
import numpy as np
import jax
import jax.numpy as jnp
from jax.experimental import pallas as pl

_STRIDE = 16
_SIZES = (128.0, 256.0, 512.0)
_RATIOS = (0.5, 1.0, 2.0)
_PRE_NMS = 5000
_POST_NMS = 1000
_NMS_THRESH = 0.7
_MIN_SIZE = 1.0


def _make_anchors_np(H, W):
    scales = np.asarray(_SIZES, dtype=np.float32)
    ratios = np.asarray(_RATIOS, dtype=np.float32)
    h_r = np.sqrt(ratios)
    w_r = 1.0 / h_r
    ws = (w_r[:, None] * scales[None, :]).reshape(-1)
    hs = (h_r[:, None] * scales[None, :]).reshape(-1)
    base = np.stack([-ws, -hs, ws, hs], axis=1) * 0.5
    sx = (np.arange(W, dtype=np.float32) + 0.5) * _STRIDE
    sy = (np.arange(H, dtype=np.float32) + 0.5) * _STRIDE
    gx, gy = np.meshgrid(sx, sy)
    shifts = np.stack([gx, gy, gx, gy], axis=-1).reshape(-1, 1, 4)
    return (shifts + base[None, :, :]).reshape(-1, 4).astype(np.float32)


def _decode(anchors, deltas):
    wa = anchors[:, 2] - anchors[:, 0]
    ha = anchors[:, 3] - anchors[:, 1]
    cxa = anchors[:, 0] + 0.5 * wa
    cya = anchors[:, 1] + 0.5 * ha
    dx, dy = deltas[:, 0], deltas[:, 1]
    lim = float(np.log(1000.0 / 16.0))
    dw = jnp.minimum(deltas[:, 2], lim)
    dh = jnp.minimum(deltas[:, 3], lim)
    cx = dx * wa + cxa
    cy = dy * ha + cya
    w = jnp.exp(dw) * wa
    h = jnp.exp(dh) * ha
    return jnp.stack([cx - 0.5 * w, cy - 0.5 * h, cx + 0.5 * w, cy + 0.5 * h], axis=1)


def _clip_boxes(boxes, image_h, image_w):
    x1 = jnp.minimum(jnp.maximum(boxes[:, 0], 0.0), image_w)
    y1 = jnp.minimum(jnp.maximum(boxes[:, 1], 0.0), image_h)
    x2 = jnp.minimum(jnp.maximum(boxes[:, 2], 0.0), image_w)
    y2 = jnp.minimum(jnp.maximum(boxes[:, 3], 0.0), image_h)
    return jnp.stack([x1, y1, x2, y2], axis=1)


def _iou_one_vs_all(box, boxes):
    ix1 = jnp.maximum(box[0], boxes[:, 0])
    iy1 = jnp.maximum(box[1], boxes[:, 1])
    ix2 = jnp.minimum(box[2], boxes[:, 2])
    iy2 = jnp.minimum(box[3], boxes[:, 3])
    inter = jnp.maximum(ix2 - ix1, 0.0) * jnp.maximum(iy2 - iy1, 0.0)
    area_b = (box[2] - box[0]) * (box[3] - box[1])
    areas = (boxes[:, 2] - boxes[:, 0]) * (boxes[:, 3] - boxes[:, 1])
    return inter / jnp.maximum(area_b + areas - inter, 1e-9)


def _nms_fixed(boxes, scores, thresh, max_out):
    def body(sc, _):
        i = jnp.argmax(sc)
        best = boxes[i]
        iou = _iou_one_vs_all(best, boxes)
        sc = jnp.where(iou > thresh, -jnp.inf, sc)
        sc = sc.at[i].set(-jnp.inf)
        return sc, i
    _, keep = jax.lax.scan(body, scores, None, length=max_out)
    return keep


def _conv_head(feature, w1, b1, w_cls, b_cls, w_reg, b_reg, precision):
    C, H, W = feature.shape[1], feature.shape[2], feature.shape[3]
    x = jnp.transpose(feature[0], (1, 2, 0))
    xp = jnp.pad(x, ((1, 1), (1, 1), (0, 0)))
    y = jnp.zeros((H * W, C), jnp.float32)
    for ky in range(3):
        for kx in range(3):
            xs = xp[ky:ky + H, kx:kx + W, :].reshape(H * W, C)
            y = y + jnp.dot(xs, w1[:, :, ky, kx].T, precision=precision)
    y = jax.nn.relu(y + b1[None, :])
    conf = jnp.dot(y, w_cls[:, :, 0, 0].T, precision=precision) + b_cls[None, :]
    off = jnp.dot(y, w_reg[:, :, 0, 0].T, precision=precision) + b_reg[None, :]
    return conf.reshape(-1), off.reshape(-1, 4)


def kernel(feature, w1, b1, w_cls, b_cls, w_reg, b_reg, image_h, image_w):
    conf_flat, off_flat = _conv_head(feature, w1, b1, w_cls, b_cls, w_reg, b_reg,
                                     jax.lax.Precision.HIGHEST)
    H, W = feature.shape[2], feature.shape[3]
    anchors = jnp.asarray(_make_anchors_np(H, W))
    pre = min(int(conf_flat.shape[0]), _PRE_NMS)
    top_scores, top_idx = jax.lax.top_k(conf_flat, pre)
    prop = _decode(anchors[top_idx], off_flat[top_idx])
    prop = _clip_boxes(prop, jnp.asarray(image_h, jnp.float32), jnp.asarray(image_w, jnp.float32))
    ws_ = prop[:, 2] - prop[:, 0]
    hs_ = prop[:, 3] - prop[:, 1]
    sc = jnp.where((ws_ >= _MIN_SIZE) & (hs_ >= _MIN_SIZE), top_scores, -jnp.inf)
    keep = _nms_fixed(prop, sc, _NMS_THRESH, _POST_NMS)
    return prop[keep]


# speedup vs baseline: 15.2524x; 15.2524x over previous
import numpy as np
import jax
import jax.numpy as jnp
from jax.experimental import pallas as pl
from jax.experimental.pallas import tpu as pltpu

_STRIDE = 16
_SIZES = (128.0, 256.0, 512.0)
_RATIOS = (0.5, 1.0, 2.0)
_PRE_NMS = 5000
_POST_NMS = 1000
_NMS_THRESH = 0.7
_MIN_SIZE = 1.0

_N = 22500
_ROWS = 176
_LANES = 128
_NEG_INF = float("-inf")
_I32_MIN = -2147483648
_I32_MAX = 2147483647
_LIM = float(np.log(1000.0 / 16.0))


def _conv2d(x, w, b):
    y = jax.lax.conv_general_dilated(x, w, window_strides=(1, 1), padding='SAME',
                                     dimension_numbers=('NCHW', 'OIHW', 'NCHW'))
    return y + b[None, :, None, None]


def _make_anchors(H, W):
    scales = jnp.asarray(_SIZES, dtype=jnp.float32)
    ratios = jnp.asarray(_RATIOS, dtype=jnp.float32)
    h_r = jnp.sqrt(ratios)
    w_r = 1.0 / h_r
    ws = (w_r[:, None] * scales[None, :]).reshape(-1)
    hs = (h_r[:, None] * scales[None, :]).reshape(-1)
    base = jnp.stack([-ws, -hs, ws, hs], axis=1) * 0.5
    sx = (jnp.arange(W, dtype=jnp.float32) + 0.5) * _STRIDE
    sy = (jnp.arange(H, dtype=jnp.float32) + 0.5) * _STRIDE
    gx, gy = jnp.meshgrid(sx, sy)
    shifts = jnp.stack([gx, gy, gx, gy], axis=-1).reshape(-1, 1, 4)
    return (shifts + base[None, :, :]).reshape(-1, 4)


def _plane(v, fill):
    return jnp.concatenate(
        [v, jnp.full((_ROWS * _LANES - _N,), fill, v.dtype)]).reshape(_ROWS, _LANES)


def _tail_kernel(hw_ref, conf_ref, dx_ref, dy_ref, dw_ref, dh_ref,
                 ax1_ref, ay1_ref, ax2_ref, ay2_ref, out_ref,
                 sc_ref, x1_ref, y1_ref, x2_ref, y2_ref, ar_ref):
    img_h = hw_ref[0, 0]
    img_w = hw_ref[0, 1]
    conf = conf_ref[...]

    lin = (jax.lax.broadcasted_iota(jnp.int32, (_ROWS, _LANES), 0) * _LANES
           + jax.lax.broadcasted_iota(jnp.int32, (_ROWS, _LANES), 1))
    valid = lin < _N

    kb = jax.lax.bitcast_convert_type(conf, jnp.int32)
    key = jnp.where(kb >= 0, kb, jnp.bitwise_xor(kb, jnp.int32(0x7FFFFFFF)))
    key = jnp.where(valid, key, jnp.int32(_I32_MIN))

    def bs_body(_, lohi):
        lo, hi = lohi
        mid = (jax.lax.shift_right_arithmetic(lo, 1)
               + jax.lax.shift_right_arithmetic(hi, 1)
               + jnp.bitwise_and(jnp.bitwise_and(lo, hi), jnp.int32(1)))
        cnt = jnp.sum(jnp.where(key >= mid, jnp.int32(1), jnp.int32(0)))
        pred = cnt >= _PRE_NMS
        return (jnp.where(pred, mid, lo), jnp.where(pred, hi, mid - 1))

    lo, _hi = jax.lax.fori_loop(0, 32, bs_body,
                                (jnp.int32(_I32_MIN), jnp.int32(_I32_MAX)))
    sel = jnp.logical_and(key >= lo, valid)

    ax1 = ax1_ref[...]; ay1 = ay1_ref[...]; ax2 = ax2_ref[...]; ay2 = ay2_ref[...]
    wa = ax2 - ax1
    ha = ay2 - ay1
    cxa = ax1 + 0.5 * wa
    cya = ay1 + 0.5 * ha
    dw = jnp.minimum(dw_ref[...], _LIM)
    dh = jnp.minimum(dh_ref[...], _LIM)
    cx = dx_ref[...] * wa + cxa
    cy = dy_ref[...] * ha + cya
    w = jnp.exp(dw) * wa
    h = jnp.exp(dh) * ha
    x1 = cx - 0.5 * w
    y1 = cy - 0.5 * h
    x2 = cx + 0.5 * w
    y2 = cy + 0.5 * h
    x1 = jnp.minimum(jnp.maximum(x1, 0.0), img_w)
    y1 = jnp.minimum(jnp.maximum(y1, 0.0), img_h)
    x2 = jnp.minimum(jnp.maximum(x2, 0.0), img_w)
    y2 = jnp.minimum(jnp.maximum(y2, 0.0), img_h)

    ws_ = x2 - x1
    hs_ = y2 - y1
    size_ok = jnp.logical_and(ws_ >= _MIN_SIZE, hs_ >= _MIN_SIZE)
    sc = jnp.where(jnp.logical_and(sel, size_ok), conf, _NEG_INF)
    areas = ws_ * hs_

    sc_ref[...] = sc
    x1_ref[...] = x1
    y1_ref[...] = y1
    x2_ref[...] = x2
    y2_ref[...] = y2
    ar_ref[...] = areas

    big = jnp.int32(_ROWS * _LANES)
    lane = jax.lax.broadcasted_iota(jnp.int32, (1, _LANES), 1)

    key_sel = jnp.where(sel, key, jnp.int32(_I32_MIN))
    m0 = jnp.max(key_sel)
    bi0 = jnp.min(jnp.where(key_sel == m0, lin, big))
    oh0 = lin == bi0
    b0x1 = jnp.sum(jnp.where(oh0, x1, 0.0))
    b0y1 = jnp.sum(jnp.where(oh0, y1, 0.0))
    b0x2 = jnp.sum(jnp.where(oh0, x2, 0.0))
    b0y2 = jnp.sum(jnp.where(oh0, y2, 0.0))
    row0 = jnp.where(lane == 0, b0x1,
           jnp.where(lane == 1, b0y1,
           jnp.where(lane == 2, b0x2,
           jnp.where(lane == 3, b0y2, 0.0))))

    def nms_body(i, _):
        scv = sc_ref[...]
        m = jnp.max(scv)

        @pl.when(m > _NEG_INF)
        def _():
            mb = scv == m
            bi = jnp.min(jnp.where(mb, lin, big))
            oh = lin == bi
            bx1 = jnp.sum(jnp.where(oh, x1_ref[...], 0.0))
            by1 = jnp.sum(jnp.where(oh, y1_ref[...], 0.0))
            bx2 = jnp.sum(jnp.where(oh, x2_ref[...], 0.0))
            by2 = jnp.sum(jnp.where(oh, y2_ref[...], 0.0))
            barea = jnp.sum(jnp.where(oh, ar_ref[...], 0.0))
            ix1 = jnp.maximum(bx1, x1_ref[...])
            iy1 = jnp.maximum(by1, y1_ref[...])
            ix2 = jnp.minimum(bx2, x2_ref[...])
            iy2 = jnp.minimum(by2, y2_ref[...])
            inter = jnp.maximum(ix2 - ix1, 0.0) * jnp.maximum(iy2 - iy1, 0.0)
            iou = inter / jnp.maximum(barea + ar_ref[...] - inter, 1e-9)
            ns = jnp.where(iou > _NMS_THRESH, _NEG_INF, scv)
            ns = jnp.where(oh, _NEG_INF, ns)
            sc_ref[...] = ns
            rowv = jnp.where(lane == 0, bx1,
                   jnp.where(lane == 1, by1,
                   jnp.where(lane == 2, bx2,
                   jnp.where(lane == 3, by2, 0.0))))
            out_ref[pl.ds(i, 1), :] = rowv

        @pl.when(jnp.logical_not(m > _NEG_INF))
        def _():
            out_ref[pl.ds(i, 1), :] = row0

        return 0

    jax.lax.fori_loop(0, _POST_NMS, nms_body, 0)


def _run_tail(hw, conf, dx, dy, dw, dh, ax1, ay1, ax2, ay2, interpret=False):
    out = pl.pallas_call(
        _tail_kernel,
        out_shape=jax.ShapeDtypeStruct((1024, _LANES), jnp.float32),
        in_specs=[pl.BlockSpec(memory_space=pltpu.SMEM)]
                 + [pl.BlockSpec(memory_space=pltpu.ANY if False else pltpu.VMEM)] * 9,
        out_specs=pl.BlockSpec(memory_space=pltpu.VMEM),
        scratch_shapes=[pltpu.VMEM((_ROWS, _LANES), jnp.float32)] * 6,
        interpret=interpret,
    )(hw, conf, dx, dy, dw, dh, ax1, ay1, ax2, ay2)
    return out


def kernel(feature, w1, b1, w_cls, b_cls, w_reg, b_reg, image_h, image_w,
           interpret=False):
    x = jax.nn.relu(_conv2d(feature, w1, b1))
    conf = _conv2d(x, w_cls, b_cls)
    off = _conv2d(x, w_reg, b_reg)
    conf_flat = jnp.transpose(conf, (0, 2, 3, 1)).reshape(-1)
    off_flat = jnp.transpose(off, (0, 2, 3, 1)).reshape(-1, 4)

    H, W = feature.shape[2], feature.shape[3]
    anchors = _make_anchors(H, W)

    hw = jnp.stack([jnp.asarray(image_h, jnp.float32),
                    jnp.asarray(image_w, jnp.float32)]).reshape(1, 2)
    conf_p = _plane(conf_flat, _NEG_INF)
    dx_p = _plane(off_flat[:, 0], 0.0)
    dy_p = _plane(off_flat[:, 1], 0.0)
    dw_p = _plane(off_flat[:, 2], 0.0)
    dh_p = _plane(off_flat[:, 3], 0.0)
    ax1_p = _plane(anchors[:, 0], 0.0)
    ay1_p = _plane(anchors[:, 1], 0.0)
    ax2_p = _plane(anchors[:, 2], 0.0)
    ay2_p = _plane(anchors[:, 3], 0.0)

    out = _run_tail(hw, conf_p, dx_p, dy_p, dw_p, dh_p,
                    ax1_p, ay1_p, ax2_p, ay2_p, interpret=interpret)
    return out[:_POST_NMS, :4]


# speedup vs baseline: 16.3508x; 1.0720x over previous
import numpy as np
import jax
import jax.numpy as jnp
from jax.experimental import pallas as pl
from jax.experimental.pallas import tpu as pltpu

_STRIDE = 16
_SIZES = (128.0, 256.0, 512.0)
_RATIOS = (0.5, 1.0, 2.0)
_PRE_NMS = 5000
_POST_NMS = 1000
_NMS_THRESH = 0.7
_MIN_SIZE = 1.0

_N = 22500
_ROWS = 176
_LANES = 128
_NEG_INF = float("-inf")
_I32_MIN = -2147483648
_I32_MAX = 2147483647
_LIM = float(np.log(1000.0 / 16.0))


def _conv2d(x, w, b):
    y = jax.lax.conv_general_dilated(x, w, window_strides=(1, 1), padding='SAME',
                                     dimension_numbers=('NCHW', 'OIHW', 'NCHW'))
    return y + b[None, :, None, None]


def _make_anchors(H, W):
    scales = jnp.asarray(_SIZES, dtype=jnp.float32)
    ratios = jnp.asarray(_RATIOS, dtype=jnp.float32)
    h_r = jnp.sqrt(ratios)
    w_r = 1.0 / h_r
    ws = (w_r[:, None] * scales[None, :]).reshape(-1)
    hs = (h_r[:, None] * scales[None, :]).reshape(-1)
    base = jnp.stack([-ws, -hs, ws, hs], axis=1) * 0.5
    sx = (jnp.arange(W, dtype=jnp.float32) + 0.5) * _STRIDE
    sy = (jnp.arange(H, dtype=jnp.float32) + 0.5) * _STRIDE
    gx, gy = jnp.meshgrid(sx, sy)
    shifts = jnp.stack([gx, gy, gx, gy], axis=-1).reshape(-1, 1, 4)
    return (shifts + base[None, :, :]).reshape(-1, 4)


def _plane(v, fill):
    return jnp.concatenate(
        [v, jnp.full((_ROWS * _LANES - _N,), fill, v.dtype)]).reshape(_ROWS, _LANES)


def _tail_kernel(hw_ref, conf_ref, dx_ref, dy_ref, dw_ref, dh_ref,
                 ax1_ref, ay1_ref, ax2_ref, ay2_ref, out_ref,
                 x1_ref, y1_ref, x2_ref, y2_ref, ar_ref):
    img_h = hw_ref[0, 0]
    img_w = hw_ref[0, 1]
    conf = conf_ref[...]

    lin = (jax.lax.broadcasted_iota(jnp.int32, (_ROWS, _LANES), 0) * _LANES
           + jax.lax.broadcasted_iota(jnp.int32, (_ROWS, _LANES), 1))
    valid = lin < _N

    kb = jax.lax.bitcast_convert_type(conf, jnp.int32)
    key = jnp.where(kb >= 0, kb, jnp.bitwise_xor(kb, jnp.int32(0x7FFFFFFF)))
    key = jnp.where(valid, key, jnp.int32(_I32_MIN))

    def bs_body(_, lohi):
        lo, hi = lohi
        mid = (jax.lax.shift_right_arithmetic(lo, 1)
               + jax.lax.shift_right_arithmetic(hi, 1)
               + jnp.bitwise_and(jnp.bitwise_and(lo, hi), jnp.int32(1)))
        cnt = jnp.sum(jnp.where(key >= mid, jnp.int32(1), jnp.int32(0)))
        pred = cnt >= _PRE_NMS
        return (jnp.where(pred, mid, lo), jnp.where(pred, hi, mid - 1))

    lo, _hi = jax.lax.fori_loop(0, 32, bs_body,
                                (jnp.int32(_I32_MIN), jnp.int32(_I32_MAX)))
    sel = jnp.logical_and(key >= lo, valid)

    ax1 = ax1_ref[...]; ay1 = ay1_ref[...]; ax2 = ax2_ref[...]; ay2 = ay2_ref[...]
    wa = ax2 - ax1
    ha = ay2 - ay1
    cxa = ax1 + 0.5 * wa
    cya = ay1 + 0.5 * ha
    dw = jnp.minimum(dw_ref[...], _LIM)
    dh = jnp.minimum(dh_ref[...], _LIM)
    cx = dx_ref[...] * wa + cxa
    cy = dy_ref[...] * ha + cya
    w = jnp.exp(dw) * wa
    h = jnp.exp(dh) * ha
    x1 = cx - 0.5 * w
    y1 = cy - 0.5 * h
    x2 = cx + 0.5 * w
    y2 = cy + 0.5 * h
    x1 = jnp.minimum(jnp.maximum(x1, 0.0), img_w)
    y1 = jnp.minimum(jnp.maximum(y1, 0.0), img_h)
    x2 = jnp.minimum(jnp.maximum(x2, 0.0), img_w)
    y2 = jnp.minimum(jnp.maximum(y2, 0.0), img_h)

    ws_ = x2 - x1
    hs_ = y2 - y1
    size_ok = jnp.logical_and(ws_ >= _MIN_SIZE, hs_ >= _MIN_SIZE)
    sc = jnp.where(jnp.logical_and(sel, size_ok), conf, _NEG_INF)
    areas = ws_ * hs_

    x1_ref[...] = x1
    y1_ref[...] = y1
    x2_ref[...] = x2
    y2_ref[...] = y2
    ar_ref[...] = areas

    big = jnp.int32(_ROWS * _LANES)
    lane = jax.lax.broadcasted_iota(jnp.int32, (1, _LANES), 1)

    key_sel = jnp.where(sel, key, jnp.int32(_I32_MIN))
    m0 = jnp.max(key_sel)
    bi0 = jnp.min(jnp.where(key_sel == m0, lin, big))
    oh0 = lin == bi0
    b0x1 = jnp.sum(jnp.where(oh0, x1, 0.0))
    b0y1 = jnp.sum(jnp.where(oh0, y1, 0.0))
    b0x2 = jnp.sum(jnp.where(oh0, x2, 0.0))
    b0y2 = jnp.sum(jnp.where(oh0, y2, 0.0))
    row0 = jnp.where(lane == 0, b0x1,
           jnp.where(lane == 1, b0y1,
           jnp.where(lane == 2, b0x2,
           jnp.where(lane == 3, b0y2, 0.0))))

    def nms_body(i, scv):
        m = jnp.max(scv)

        def live(scv):
            mb = scv == m
            bi = jnp.min(jnp.where(mb, lin, big))
            r = jax.lax.shift_right_logical(bi, 7)
            l = jnp.bitwise_and(bi, jnp.int32(_LANES - 1))
            le = lane == l
            bx1 = jnp.sum(jnp.where(le, x1_ref[pl.ds(r, 1), :], 0.0))
            by1 = jnp.sum(jnp.where(le, y1_ref[pl.ds(r, 1), :], 0.0))
            bx2 = jnp.sum(jnp.where(le, x2_ref[pl.ds(r, 1), :], 0.0))
            by2 = jnp.sum(jnp.where(le, y2_ref[pl.ds(r, 1), :], 0.0))
            barea = jnp.sum(jnp.where(le, ar_ref[pl.ds(r, 1), :], 0.0))
            oh = lin == bi
            ix1 = jnp.maximum(bx1, x1_ref[...])
            iy1 = jnp.maximum(by1, y1_ref[...])
            ix2 = jnp.minimum(bx2, x2_ref[...])
            iy2 = jnp.minimum(by2, y2_ref[...])
            inter = jnp.maximum(ix2 - ix1, 0.0) * jnp.maximum(iy2 - iy1, 0.0)
            iou = inter / jnp.maximum(barea + ar_ref[...] - inter, 1e-9)
            ns = jnp.where(iou > _NMS_THRESH, _NEG_INF, scv)
            ns = jnp.where(oh, _NEG_INF, ns)
            rowv = jnp.where(lane == 0, bx1,
                   jnp.where(lane == 1, by1,
                   jnp.where(lane == 2, bx2,
                   jnp.where(lane == 3, by2, 0.0))))
            out_ref[pl.ds(i, 1), :] = rowv
            return ns

        def dead(scv):
            out_ref[pl.ds(i, 1), :] = row0
            return scv

        return jax.lax.cond(m > _NEG_INF, live, dead, scv)

    jax.lax.fori_loop(0, _POST_NMS, nms_body, sc)


def _run_tail(hw, conf, dx, dy, dw, dh, ax1, ay1, ax2, ay2, interpret=False):
    out = pl.pallas_call(
        _tail_kernel,
        out_shape=jax.ShapeDtypeStruct((1024, _LANES), jnp.float32),
        in_specs=[pl.BlockSpec(memory_space=pltpu.SMEM)]
                 + [pl.BlockSpec(memory_space=pltpu.ANY if False else pltpu.VMEM)] * 9,
        out_specs=pl.BlockSpec(memory_space=pltpu.VMEM),
        scratch_shapes=[pltpu.VMEM((_ROWS, _LANES), jnp.float32)] * 5,
        interpret=interpret,
    )(hw, conf, dx, dy, dw, dh, ax1, ay1, ax2, ay2)
    return out


def kernel(feature, w1, b1, w_cls, b_cls, w_reg, b_reg, image_h, image_w,
           interpret=False):
    x = jax.nn.relu(_conv2d(feature, w1, b1))
    conf = _conv2d(x, w_cls, b_cls)
    off = _conv2d(x, w_reg, b_reg)
    conf_flat = jnp.transpose(conf, (0, 2, 3, 1)).reshape(-1)
    off_flat = jnp.transpose(off, (0, 2, 3, 1)).reshape(-1, 4)

    H, W = feature.shape[2], feature.shape[3]
    anchors = _make_anchors(H, W)

    hw = jnp.stack([jnp.asarray(image_h, jnp.float32),
                    jnp.asarray(image_w, jnp.float32)]).reshape(1, 2)
    conf_p = _plane(conf_flat, _NEG_INF)
    dx_p = _plane(off_flat[:, 0], 0.0)
    dy_p = _plane(off_flat[:, 1], 0.0)
    dw_p = _plane(off_flat[:, 2], 0.0)
    dh_p = _plane(off_flat[:, 3], 0.0)
    ax1_p = _plane(anchors[:, 0], 0.0)
    ay1_p = _plane(anchors[:, 1], 0.0)
    ax2_p = _plane(anchors[:, 2], 0.0)
    ay2_p = _plane(anchors[:, 3], 0.0)

    out = _run_tail(hw, conf_p, dx_p, dy_p, dw_p, dh_p,
                    ax1_p, ay1_p, ax2_p, ay2_p, interpret=interpret)
    return out[:_POST_NMS, :4]


# speedup vs baseline: 17.4633x; 1.0680x over previous
import numpy as np
import jax
import jax.numpy as jnp
from jax.experimental import pallas as pl
from jax.experimental.pallas import tpu as pltpu

_STRIDE = 16
_SIZES = (128.0, 256.0, 512.0)
_RATIOS = (0.5, 1.0, 2.0)
_PRE_NMS = 5000
_POST_NMS = 1000
_NMS_THRESH = 0.7
_MIN_SIZE = 1.0

_ROWS = 40
_LANES = 128
_NEG_INF = float("-inf")


def _conv2d(x, w, b):
    y = jax.lax.conv_general_dilated(x, w, window_strides=(1, 1), padding='SAME',
                                     dimension_numbers=('NCHW', 'OIHW', 'NCHW'))
    return y + b[None, :, None, None]


def _make_anchors(H, W):
    scales = jnp.asarray(_SIZES, dtype=jnp.float32)
    ratios = jnp.asarray(_RATIOS, dtype=jnp.float32)
    h_r = jnp.sqrt(ratios)
    w_r = 1.0 / h_r
    ws = (w_r[:, None] * scales[None, :]).reshape(-1)
    hs = (h_r[:, None] * scales[None, :]).reshape(-1)
    base = jnp.stack([-ws, -hs, ws, hs], axis=1) * 0.5
    sx = (jnp.arange(W, dtype=jnp.float32) + 0.5) * _STRIDE
    sy = (jnp.arange(H, dtype=jnp.float32) + 0.5) * _STRIDE
    gx, gy = jnp.meshgrid(sx, sy)
    shifts = jnp.stack([gx, gy, gx, gy], axis=-1).reshape(-1, 1, 4)
    return (shifts + base[None, :, :]).reshape(-1, 4)


def _decode(anchors, deltas):
    wa = anchors[:, 2] - anchors[:, 0]
    ha = anchors[:, 3] - anchors[:, 1]
    cxa = anchors[:, 0] + 0.5 * wa
    cya = anchors[:, 1] + 0.5 * ha
    dx, dy = deltas[:, 0], deltas[:, 1]
    lim = float(np.log(1000.0 / 16.0))
    dw = jnp.minimum(deltas[:, 2], lim)
    dh = jnp.minimum(deltas[:, 3], lim)
    cx = dx * wa + cxa
    cy = dy * ha + cya
    w = jnp.exp(dw) * wa
    h = jnp.exp(dh) * ha
    return jnp.stack([cx - 0.5 * w, cy - 0.5 * h, cx + 0.5 * w, cy + 0.5 * h], axis=1)


def _clip_boxes(boxes, image_h, image_w):
    x1 = jnp.minimum(jnp.maximum(boxes[:, 0], 0.0), image_w)
    y1 = jnp.minimum(jnp.maximum(boxes[:, 1], 0.0), image_h)
    x2 = jnp.minimum(jnp.maximum(boxes[:, 2], 0.0), image_w)
    y2 = jnp.minimum(jnp.maximum(boxes[:, 3], 0.0), image_h)
    return jnp.stack([x1, y1, x2, y2], axis=1)


def _plane(v, fill):
    return jnp.concatenate(
        [v, jnp.full((_ROWS * _LANES - _PRE_NMS,), fill, v.dtype)]).reshape(_ROWS, _LANES)


def _nms_kernel(sc_ref, x1_ref, y1_ref, x2_ref, y2_ref, out_ref, ar_ref):
    x1 = x1_ref[...]
    y1 = y1_ref[...]
    x2 = x2_ref[...]
    y2 = y2_ref[...]
    ws_ = x2 - x1
    hs_ = y2 - y1
    size_ok = jnp.logical_and(ws_ >= _MIN_SIZE, hs_ >= _MIN_SIZE)
    sc = jnp.where(size_ok, sc_ref[...], _NEG_INF)
    ar_ref[...] = ws_ * hs_

    lin = (jax.lax.broadcasted_iota(jnp.int32, (_ROWS, _LANES), 0) * _LANES
           + jax.lax.broadcasted_iota(jnp.int32, (_ROWS, _LANES), 1))
    big = jnp.int32(_ROWS * _LANES)
    lane = jax.lax.broadcasted_iota(jnp.int32, (1, _LANES), 1)

    oh0 = lin == 0
    b0x1 = jnp.sum(jnp.where(oh0, x1, 0.0))
    b0y1 = jnp.sum(jnp.where(oh0, y1, 0.0))
    b0x2 = jnp.sum(jnp.where(oh0, x2, 0.0))
    b0y2 = jnp.sum(jnp.where(oh0, y2, 0.0))
    row0 = jnp.where(lane == 0, b0x1,
           jnp.where(lane == 1, b0y1,
           jnp.where(lane == 2, b0x2,
           jnp.where(lane == 3, b0y2, 0.0))))

    def nms_body(i, scv):
        m = jnp.max(scv)

        def live(scv):
            mb = scv == m
            bi = jnp.min(jnp.where(mb, lin, big))
            r = jax.lax.shift_right_logical(bi, 7)
            l = jnp.bitwise_and(bi, jnp.int32(_LANES - 1))
            le = lane == l
            bx1 = jnp.sum(jnp.where(le, x1_ref[pl.ds(r, 1), :], 0.0))
            by1 = jnp.sum(jnp.where(le, y1_ref[pl.ds(r, 1), :], 0.0))
            bx2 = jnp.sum(jnp.where(le, x2_ref[pl.ds(r, 1), :], 0.0))
            by2 = jnp.sum(jnp.where(le, y2_ref[pl.ds(r, 1), :], 0.0))
            barea = jnp.sum(jnp.where(le, ar_ref[pl.ds(r, 1), :], 0.0))
            oh = lin == bi
            ix1 = jnp.maximum(bx1, x1_ref[...])
            iy1 = jnp.maximum(by1, y1_ref[...])
            ix2 = jnp.minimum(bx2, x2_ref[...])
            iy2 = jnp.minimum(by2, y2_ref[...])
            inter = jnp.maximum(ix2 - ix1, 0.0) * jnp.maximum(iy2 - iy1, 0.0)
            iou = inter / jnp.maximum(barea + ar_ref[...] - inter, 1e-9)
            ns = jnp.where(iou > _NMS_THRESH, _NEG_INF, scv)
            ns = jnp.where(oh, _NEG_INF, ns)
            rowv = jnp.where(lane == 0, bx1,
                   jnp.where(lane == 1, by1,
                   jnp.where(lane == 2, bx2,
                   jnp.where(lane == 3, by2, 0.0))))
            out_ref[pl.ds(i, 1), :] = rowv
            return ns

        def dead(scv):
            out_ref[pl.ds(i, 1), :] = row0
            return scv

        return jax.lax.cond(m > _NEG_INF, live, dead, scv)

    jax.lax.fori_loop(0, _POST_NMS, nms_body, sc)


def _run_nms(sc, x1, y1, x2, y2, interpret=False):
    return pl.pallas_call(
        _nms_kernel,
        out_shape=jax.ShapeDtypeStruct((1024, _LANES), jnp.float32),
        in_specs=[pl.BlockSpec(memory_space=pltpu.VMEM)] * 5,
        out_specs=pl.BlockSpec(memory_space=pltpu.VMEM),
        scratch_shapes=[pltpu.VMEM((_ROWS, _LANES), jnp.float32)],
        interpret=interpret,
    )(sc, x1, y1, x2, y2)


def kernel(feature, w1, b1, w_cls, b_cls, w_reg, b_reg, image_h, image_w,
           interpret=False):
    x = jax.nn.relu(_conv2d(feature, w1, b1))
    conf = _conv2d(x, w_cls, b_cls)
    off = _conv2d(x, w_reg, b_reg)
    conf_flat = jnp.transpose(conf, (0, 2, 3, 1)).reshape(-1)
    off_flat = jnp.transpose(off, (0, 2, 3, 1)).reshape(-1, 4)

    H, W = feature.shape[2], feature.shape[3]
    anchors = _make_anchors(H, W)
    pre = min(int(conf_flat.shape[0]), _PRE_NMS)
    top_scores, top_idx = jax.lax.top_k(conf_flat, pre)
    prop = _decode(anchors[top_idx], off_flat[top_idx])
    prop = _clip_boxes(prop, jnp.asarray(image_h, jnp.float32),
                       jnp.asarray(image_w, jnp.float32))

    sc_p = _plane(top_scores, _NEG_INF)
    x1_p = _plane(prop[:, 0], 0.0)
    y1_p = _plane(prop[:, 1], 0.0)
    x2_p = _plane(prop[:, 2], 0.0)
    y2_p = _plane(prop[:, 3], 0.0)

    if interpret:
        out = _run_nms(sc_p, x1_p, y1_p, x2_p, y2_p, interpret=True)
    else:
        out = jax.lax.cond(
            top_scores[0] >= -3.0e38,
            lambda: _run_nms(sc_p, x1_p, y1_p, x2_p, y2_p),
            lambda: jnp.zeros((1024, _LANES), jnp.float32))
    return out[:_POST_NMS, :4]
